# baseline (speedup 1.0000x reference)
import math

import jax
import jax.numpy as jnp
from jax import lax
from jax.experimental import pallas as pl
from jax.experimental.pallas import tpu as pltpu

_EP = 8


def _build(B, L, N, D, H, E):
    TB = L
    S = B * L * N
    G = S // TB
    GB = G // B
    cap = int(math.ceil(S / E * 1.0))
    capf = float(cap)
    LN = float(L * N)

    def body(feat_ref, gates_ref, we_ref, bep_ref, wp_ref, bp_ref, nm_ref,
             pooled_ref, aux_ref, z_ref, stats_ref, base_ref):
        g = pl.program_id(0)

        @pl.when(g == 0)
        def _init():
            z_ref[...] = jnp.zeros_like(z_ref)
            stats_ref[...] = jnp.zeros_like(stats_ref)
            base_ref[...] = jnp.zeros_like(base_ref)

        x = feat_ref[0]
        gates = gates_ref[...]
        lane = lax.broadcasted_iota(jnp.int32, (TB, _EP), 1)
        m = jnp.max(gates, axis=1, keepdims=True)
        idxv = jnp.min(jnp.where(gates >= m, lane, _EP), axis=1, keepdims=True)
        onehot = (lane == idxv).astype(jnp.float32)
        r = lax.broadcasted_iota(jnp.int32, (TB, TB), 0)
        c = lax.broadcasted_iota(jnp.int32, (TB, TB), 1)
        tri = (r >= c).astype(jnp.float32)
        csum = jnp.dot(tri, onehot, preferred_element_type=jnp.float32)
        base = base_ref[...]
        keep = ((base + csum - 1.0) < capf).astype(jnp.float32)
        w = gates * onehot * keep
        base_ref[...] = base + jnp.sum(onehot, axis=0, keepdims=True)
        stats_ref[0:1, :] += jnp.sum(gates, axis=0, keepdims=True)
        stats_ref[1:2, :] += jnp.sum(onehot, axis=0, keepdims=True)
        wrow = jnp.sum(w, axis=0, keepdims=True)
        zpart = lax.dot_general(w, x, (((0,), (0,)), ((), ())),
                                preferred_element_type=jnp.float32,
                                precision=lax.Precision.HIGHEST)

        @pl.when(g < GB)
        def _acc0():
            z_ref[0:_EP, :] += zpart
            stats_ref[2:3, :] += wrow

        @pl.when(g >= GB)
        def _acc1():
            z_ref[_EP:2 * _EP, :] += zpart
            stats_ref[3:4, :] += wrow

        @pl.when(g == G - 1)
        def _finish():
            acc = jnp.dot(stats_ref[2:4, :], bep_ref[...],
                          preferred_element_type=jnp.float32,
                          precision=lax.Precision.HIGHEST)
            for e in range(E):
                ze = jnp.concatenate(
                    [z_ref[e:e + 1, :], z_ref[_EP + e:_EP + e + 1, :]], axis=0)
                acc = acc + lax.dot_general(
                    ze, we_ref[e], (((1,), (1,)), ((), ())),
                    preferred_element_type=jnp.float32,
                    precision=lax.Precision.HIGHEST)
            proj = jnp.dot(acc, wp_ref[...],
                           preferred_element_type=jnp.float32,
                           precision=lax.Precision.HIGHEST)
            proj = proj + LN * bp_ref[...]
            count = jnp.maximum(jnp.sum(nm_ref[...], axis=1, keepdims=True), 1.0)
            pooled_ref[...] = proj / count
            aux = jnp.sum(stats_ref[0:1, :] * stats_ref[1:2, :]) * (E / (S * S))
            aux_ref[...] = jnp.full((1, _EP), aux, jnp.float32)

    call = pl.pallas_call(
        body,
        grid=(G,),
        in_specs=[
            pl.BlockSpec((1, L, D), lambda g: (g // GB, 0, g % GB)),
            pl.BlockSpec((TB, _EP), lambda g: (g, 0)),
            pl.BlockSpec((E, D, D), lambda g: (0, 0, 0)),
            pl.BlockSpec((_EP, D), lambda g: (0, 0)),
            pl.BlockSpec((D, H), lambda g: (0, 0)),
            pl.BlockSpec((1, H), lambda g: (0, 0)),
            pl.BlockSpec((B, L * N), lambda g: (0, 0)),
        ],
        out_specs=[
            pl.BlockSpec((B, H), lambda g: (0, 0)),
            pl.BlockSpec((1, _EP), lambda g: (0, 0)),
        ],
        out_shape=[
            jax.ShapeDtypeStruct((B, H), jnp.float32),
            jax.ShapeDtypeStruct((1, _EP), jnp.float32),
        ],
        scratch_shapes=[
            pltpu.VMEM((2 * _EP, D), jnp.float32),
            pltpu.VMEM((8, _EP), jnp.float32),
            pltpu.VMEM((1, _EP), jnp.float32),
        ],
        compiler_params=pltpu.CompilerParams(
            dimension_semantics=("arbitrary",)),
    )
    return call


def kernel(features, mask, Wg, We, be, Wp, bp):
    B, L, N, D = features.shape
    E = Wg.shape[1]
    H = Wp.shape[1]
    logits = features.reshape(B * L * N, D) @ Wg
    logits = logits.reshape(B, L, N, E).transpose(0, 2, 1, 3).reshape(
        B * N * L, E)
    gates = jax.nn.softmax(logits, axis=1)
    gates8 = jnp.pad(gates, ((0, 0), (0, _EP - E)))
    feat3 = features.reshape(B, L, N * D)
    bep = jnp.pad(be.astype(jnp.float32), ((0, _EP - E), (0, 0)))
    bp2 = bp.reshape(1, H).astype(jnp.float32)
    nm = jnp.logical_not(mask).reshape(B, L * N).astype(jnp.float32)
    call = _build(B, L, N, D, H, E)
    pooled, aux = call(feat3, gates8, We, bep, Wp, bp2, nm)
    return pooled, aux[0, 0]


# speedup vs baseline: 1.2702x; 1.2702x over previous
import math

import jax
import jax.numpy as jnp
from jax import lax
from jax.experimental import pallas as pl
from jax.experimental.pallas import tpu as pltpu

_EP = 8


def _build(B, L, N, D, H, E):
    TB = L
    S = B * L * N
    G = S // TB
    GB = G // B
    cap = int(math.ceil(S / E * 1.0))
    capf = float(cap)
    LN = float(L * N)

    def body(feat_ref, gates_ref, we_ref, bep_ref, wp_ref, bp_ref, nm_ref,
             pooled_ref, aux_ref, z_ref, stats_ref, base_ref):
        g = pl.program_id(0)

        @pl.when(g == 0)
        def _init():
            z_ref[...] = jnp.zeros_like(z_ref)
            stats_ref[...] = jnp.zeros_like(stats_ref)
            base_ref[...] = jnp.zeros_like(base_ref)

        x = feat_ref[0]
        gates = gates_ref[...]
        lane = lax.broadcasted_iota(jnp.int32, (TB, _EP), 1)
        m = jnp.max(gates, axis=1, keepdims=True)
        idxv = jnp.min(jnp.where(gates >= m, lane, _EP), axis=1, keepdims=True)
        onehot = (lane == idxv).astype(jnp.float32)
        r = lax.broadcasted_iota(jnp.int32, (TB, TB), 0)
        c = lax.broadcasted_iota(jnp.int32, (TB, TB), 1)
        tri = (r >= c).astype(jnp.float32)
        csum = jnp.dot(tri, onehot, preferred_element_type=jnp.float32)
        base = base_ref[...]
        keep = ((base + csum - 1.0) < capf).astype(jnp.float32)
        w = gates * onehot * keep
        base_ref[...] = base + jnp.sum(onehot, axis=0, keepdims=True)
        stats_ref[0:1, :] += jnp.sum(gates, axis=0, keepdims=True)
        stats_ref[1:2, :] += jnp.sum(onehot, axis=0, keepdims=True)
        wrow = jnp.sum(w, axis=0, keepdims=True)
        zpart = lax.dot_general(w, x, (((0,), (0,)), ((), ())),
                                preferred_element_type=jnp.float32,
                                precision=lax.Precision.DEFAULT)

        @pl.when(g < GB)
        def _acc0():
            z_ref[0:_EP, :] += zpart
            stats_ref[2:3, :] += wrow

        @pl.when(g >= GB)
        def _acc1():
            z_ref[_EP:2 * _EP, :] += zpart
            stats_ref[3:4, :] += wrow

        @pl.when(g == G - 1)
        def _finish():
            acc = jnp.dot(stats_ref[2:4, :], bep_ref[...],
                          preferred_element_type=jnp.float32,
                          precision=lax.Precision.DEFAULT)
            for e in range(E):
                ze = jnp.concatenate(
                    [z_ref[e:e + 1, :], z_ref[_EP + e:_EP + e + 1, :]], axis=0)
                acc = acc + lax.dot_general(
                    ze, we_ref[e], (((1,), (1,)), ((), ())),
                    preferred_element_type=jnp.float32,
                    precision=lax.Precision.DEFAULT)
            proj = jnp.dot(acc, wp_ref[...],
                           preferred_element_type=jnp.float32,
                           precision=lax.Precision.DEFAULT)
            proj = proj + LN * bp_ref[...]
            count = jnp.maximum(jnp.sum(nm_ref[...], axis=1, keepdims=True), 1.0)
            pooled_ref[...] = proj / count
            aux = jnp.sum(stats_ref[0:1, :] * stats_ref[1:2, :]) * (E / (S * S))
            aux_ref[...] = jnp.full((1, _EP), aux, jnp.float32)

    call = pl.pallas_call(
        body,
        grid=(G,),
        in_specs=[
            pl.BlockSpec((1, L, D), lambda g: (g // GB, 0, g % GB)),
            pl.BlockSpec((TB, _EP), lambda g: (g, 0)),
            pl.BlockSpec((E, D, D), lambda g: (0, 0, 0)),
            pl.BlockSpec((_EP, D), lambda g: (0, 0)),
            pl.BlockSpec((D, H), lambda g: (0, 0)),
            pl.BlockSpec((1, H), lambda g: (0, 0)),
            pl.BlockSpec((B, L * N), lambda g: (0, 0)),
        ],
        out_specs=[
            pl.BlockSpec((B, H), lambda g: (0, 0)),
            pl.BlockSpec((1, _EP), lambda g: (0, 0)),
        ],
        out_shape=[
            jax.ShapeDtypeStruct((B, H), jnp.float32),
            jax.ShapeDtypeStruct((1, _EP), jnp.float32),
        ],
        scratch_shapes=[
            pltpu.VMEM((2 * _EP, D), jnp.float32),
            pltpu.VMEM((8, _EP), jnp.float32),
            pltpu.VMEM((1, _EP), jnp.float32),
        ],
        compiler_params=pltpu.CompilerParams(
            dimension_semantics=("arbitrary",)),
    )
    return call


def kernel(features, mask, Wg, We, be, Wp, bp):
    B, L, N, D = features.shape
    E = Wg.shape[1]
    H = Wp.shape[1]
    x2d = jnp.transpose(features, (0, 2, 1, 3)).reshape(B * N * L, D)
    gates = jax.nn.softmax(x2d @ Wg, axis=1)
    gates8 = jnp.pad(gates, ((0, 0), (0, _EP - E)))
    feat3 = features.reshape(B, L, N * D)
    bep = jnp.pad(be.astype(jnp.float32), ((0, _EP - E), (0, 0)))
    bp2 = bp.reshape(1, H).astype(jnp.float32)
    nm = jnp.logical_not(mask).reshape(B, L * N).astype(jnp.float32)
    call = _build(B, L, N, D, H, E)
    pooled, aux = call(feat3, gates8, We, bep, Wp, bp2, nm)
    return pooled, aux[0, 0]
